# baseline (speedup 1.0000x reference)
import functools

import jax
import jax.numpy as jnp
from jax import lax
from jax.experimental import pallas as pl
from jax.experimental.pallas import tpu as pltpu
from jax.experimental.pallas import tpu_sc as plsc

N = 10000
E = 320000
D = 128
DE = 16
TT = 3

NC = 2
NS = 16
NW = NC * NS

CHUNK = 128
NSTEPS = E // CHUNK
SPT_MAX = 80

NX = 10112
NF = 30080
RX = NX // NS
RF = NF // NS

BN = 1000
NB = N // BN


def _chunks(total, step):
    out = []
    off = 0
    while off < total:
        out.append((off, min(step, total - off)))
        off += step
    return out


def _tile_steps(wid):
    base = 78 * wid + 2 * jnp.minimum(wid, 2)
    nsteps = jnp.where(wid < 2, 80, 78)
    return base, nsteps


def _node_msg_body(x_ref, m_ref, w0_ref, w1_ref, wx_ref, y_ref):
    xb = x_ref[...]
    a0 = jnp.dot(xb, w0_ref[...], preferred_element_type=jnp.float32)
    a1 = jnp.dot(xb, w1_ref[...], preferred_element_type=jnp.float32)
    x2 = a0 + m_ref[...] * (a1 - a0)
    y_ref[...] = jnp.dot(x2, wx_ref[0], preferred_element_type=jnp.float32)


def _node_msg(x, m, w0t, w1t, wxt):
    return pl.pallas_call(
        _node_msg_body,
        grid=(NB, TT),
        in_specs=[
            pl.BlockSpec((BN, D), lambda i, t: (i, 0)),
            pl.BlockSpec((BN, 1), lambda i, t: (i, 0)),
            pl.BlockSpec((D, D), lambda i, t: (0, 0)),
            pl.BlockSpec((D, D), lambda i, t: (0, 0)),
            pl.BlockSpec((1, D, D), lambda i, t: (t, 0, 0)),
        ],
        out_specs=pl.BlockSpec((BN, D), lambda i, t: (t * NB + i, 0)),
        out_shape=jax.ShapeDtypeStruct((TT * N, D), jnp.float32),
    )(x, m, w0t, w1t, wxt)


@functools.partial(
    pl.kernel,
    out_type=jax.ShapeDtypeStruct((NC, NX, D), jnp.float32),
    mesh=plsc.VectorSubcoreMesh(core_axis_name="c", subcore_axis_name="s"),
    compiler_params=pltpu.CompilerParams(use_tc_tiling_on_sc=False),
    scratch_types=[
        pltpu.VMEM_SHARED((NX, D), jnp.float32),
        pltpu.VMEM((SPT_MAX, CHUNK), jnp.int32),
        pltpu.VMEM((1, CHUNK), jnp.int32),
        pltpu.VMEM((1, CHUNK), jnp.int32),
        pltpu.VMEM((CHUNK, D), jnp.float32),
        pltpu.VMEM((CHUNK, D), jnp.float32),
        pltpu.SemaphoreType.DMA,
        pltpu.SemaphoreType.DMA,
    ],
)
def _sc_scatter_x(y_hbm, gidx_hbm, dst_hbm, px_hbm,
                  accx, gidx_v, dstA, dstB, rowsA, rowsB, semA, semB):
    cid = lax.axis_index("c")
    sid = lax.axis_index("s")
    wid = cid * NS + sid
    base, nsteps = _tile_steps(wid)

    def _zrow(i, c):
        for j in range(D // 16):
            rowsA[i, pl.ds(j * 16, 16)] = jnp.zeros((16,), jnp.float32)
        return c
    lax.fori_loop(0, CHUNK, _zrow, 0)
    bx = sid * RX
    for off, nr in _chunks(RX, CHUNK):
        pltpu.sync_copy(rowsA.at[pl.ds(0, nr)], accx.at[pl.ds(bx + off, nr)])
    plsc.subcore_barrier()

    pltpu.sync_copy(gidx_hbm.at[pl.ds(base, 78)], gidx_v.at[pl.ds(0, 78)])

    @pl.when(wid < 2)
    def _():
        pltpu.sync_copy(gidx_hbm.at[pl.ds(base + 78, 2)], gidx_v.at[pl.ds(78, 2)])

    def _waitA():
        pltpu.make_async_copy(y_hbm.at[gidx_v.at[0]], rowsA, semA).wait()

    def _waitB():
        pltpu.make_async_copy(y_hbm.at[gidx_v.at[0]], rowsB, semB).wait()

    pltpu.sync_copy(dst_hbm.at[base], dstA.at[0])
    pltpu.async_copy(y_hbm.at[gidx_v.at[0]], rowsA, semA)

    def _pair(jj, c):
        s1 = 2 * jj + 1
        pltpu.sync_copy(dst_hbm.at[base + s1], dstB.at[0])
        pltpu.async_copy(y_hbm.at[gidx_v.at[s1]], rowsB, semB)
        _waitA()
        pltpu.sync_copy(rowsA, accx.at[dstA.at[0]], add=True)
        s2 = 2 * jj + 2
        pltpu.sync_copy(dst_hbm.at[base + s2], dstA.at[0])
        pltpu.async_copy(y_hbm.at[gidx_v.at[s2]], rowsA, semA)
        _waitB()
        pltpu.sync_copy(rowsB, accx.at[dstB.at[0]], add=True)
        return c
    lax.fori_loop(0, nsteps // 2 - 1, _pair, 0, unroll=False)

    pltpu.sync_copy(dst_hbm.at[base + nsteps - 1], dstB.at[0])
    pltpu.async_copy(y_hbm.at[gidx_v.at[nsteps - 1]], rowsB, semB)
    _waitA()
    pltpu.sync_copy(rowsA, accx.at[dstA.at[0]], add=True)
    _waitB()
    pltpu.sync_copy(rowsB, accx.at[dstB.at[0]], add=True)
    plsc.subcore_barrier()

    for off, nr in _chunks(RX, CHUNK):
        pltpu.sync_copy(accx.at[pl.ds(bx + off, nr)], rowsA.at[pl.ds(0, nr)])
        pltpu.sync_copy(rowsA.at[pl.ds(0, nr)], px_hbm.at[cid, pl.ds(bx + off, nr)])


@functools.partial(
    pl.kernel,
    out_type=jax.ShapeDtypeStruct((NC, NF, DE), jnp.float32),
    mesh=plsc.VectorSubcoreMesh(core_axis_name="c", subcore_axis_name="s"),
    compiler_params=pltpu.CompilerParams(use_tc_tiling_on_sc=False),
    scratch_types=[
        pltpu.VMEM_SHARED((NF, DE), jnp.float32),
        pltpu.VMEM((SPT_MAX, CHUNK), jnp.int32),
        pltpu.VMEM((CHUNK, DE), jnp.float32),
        pltpu.VMEM((CHUNK, DE), jnp.float32),
        pltpu.SemaphoreType.DMA,
        pltpu.SemaphoreType.DMA,
    ],
)
def _sc_scatter_f(ef_hbm, sidx_hbm, pf_hbm,
                  accf, sidx_v, efA, efB, semA, semB):
    cid = lax.axis_index("c")
    sid = lax.axis_index("s")
    wid = cid * NS + sid
    base, nsteps = _tile_steps(wid)

    def _zrow(i, c):
        efA[i] = jnp.zeros((16,), jnp.float32)
        return c
    lax.fori_loop(0, CHUNK, _zrow, 0)
    bf = sid * RF
    for off, nr in _chunks(RF, CHUNK):
        pltpu.sync_copy(efA.at[pl.ds(0, nr)], accf.at[pl.ds(bf + off, nr)])
    plsc.subcore_barrier()

    pltpu.sync_copy(sidx_hbm.at[pl.ds(base, 78)], sidx_v.at[pl.ds(0, 78)])

    @pl.when(wid < 2)
    def _():
        pltpu.sync_copy(sidx_hbm.at[pl.ds(base + 78, 2)], sidx_v.at[pl.ds(78, 2)])

    def _waitA():
        pltpu.make_async_copy(ef_hbm.at[pl.ds(0, CHUNK)], efA, semA).wait()

    def _waitB():
        pltpu.make_async_copy(ef_hbm.at[pl.ds(0, CHUNK)], efB, semB).wait()

    ebase = base * CHUNK
    pltpu.async_copy(ef_hbm.at[pl.ds(ebase, CHUNK)], efA, semA)

    def _pair(jj, c):
        s1 = 2 * jj + 1
        pltpu.async_copy(ef_hbm.at[pl.ds(ebase + s1 * CHUNK, CHUNK)], efB, semB)
        _waitA()
        pltpu.sync_copy(efA, accf.at[sidx_v.at[2 * jj]], add=True)
        s2 = 2 * jj + 2
        pltpu.async_copy(ef_hbm.at[pl.ds(ebase + s2 * CHUNK, CHUNK)], efA, semA)
        _waitB()
        pltpu.sync_copy(efB, accf.at[sidx_v.at[s1]], add=True)
        return c
    lax.fori_loop(0, nsteps // 2 - 1, _pair, 0, unroll=False)

    pltpu.async_copy(ef_hbm.at[pl.ds(ebase + (nsteps - 1) * CHUNK, CHUNK)], efB, semB)
    _waitA()
    pltpu.sync_copy(efA, accf.at[sidx_v.at[nsteps - 2]], add=True)
    _waitB()
    pltpu.sync_copy(efB, accf.at[sidx_v.at[nsteps - 1]], add=True)
    plsc.subcore_barrier()

    for off, nr in _chunks(RF, CHUNK):
        pltpu.sync_copy(accf.at[pl.ds(bf + off, nr)], efA.at[pl.ds(0, nr)])
        pltpu.sync_copy(efA.at[pl.ds(0, nr)], pf_hbm.at[cid, pl.ds(bf + off, nr)])


def _combine_body(px_ref, pf0_ref, pf1_ref, pf2_ref, wf_ref, o_ref):
    o = px_ref[0] + px_ref[1]
    for t, pf_ref in enumerate((pf0_ref, pf1_ref, pf2_ref)):
        s = pf_ref[0] + pf_ref[1]
        o = o + jnp.dot(s, wf_ref[t], preferred_element_type=jnp.float32)
    o_ref[...] = o


def _combine(px, pf, wft):
    return pl.pallas_call(
        _combine_body,
        grid=(NB,),
        in_specs=[
            pl.BlockSpec((NC, BN, D), lambda i: (0, i, 0)),
            pl.BlockSpec((NC, BN, DE), lambda i: (0, i, 0)),
            pl.BlockSpec((NC, BN, DE), lambda i: (0, i + NB, 0)),
            pl.BlockSpec((NC, BN, DE), lambda i: (0, i + 2 * NB, 0)),
            pl.BlockSpec((TT, DE, D), lambda i: (0, 0, 0)),
        ],
        out_specs=pl.BlockSpec((BN, D), lambda i: (i, 0)),
        out_shape=jax.ShapeDtypeStruct((N, D), jnp.float32),
    )(px, pf, pf, pf, wft)


def kernel(x, edge_index, edge_feature, node_type, edge_type, W_node, W_msg):
    w0t = W_node[0].T
    w1t = W_node[1].T
    wxt = jnp.transpose(W_msg[:, :, :D], (0, 2, 1))
    wft = jnp.transpose(W_msg[:, :, D:], (0, 2, 1))
    m = node_type.astype(jnp.float32)[:, None]

    y = _node_msg(x, m, w0t, w1t, wxt)

    src = edge_index[0]
    dst = edge_index[1]
    gidx = (edge_type * N + src).reshape(NSTEPS, CHUNK)
    dstp = dst.reshape(NSTEPS, CHUNK)
    sidxp = (edge_type * N + dst).reshape(NSTEPS, CHUNK)

    ef_b, y_b = lax.optimization_barrier((edge_feature, y))
    px = _sc_scatter_x(y_b, gidx, dstp)
    sidx_b, px_b = lax.optimization_barrier((sidxp, px))
    pf = _sc_scatter_f(ef_b, sidx_b)
    return _combine(px_b, pf, wft)


# speedup vs baseline: 1.0055x; 1.0055x over previous
import functools

import jax
import jax.numpy as jnp
from jax import lax
from jax.experimental import pallas as pl
from jax.experimental.pallas import tpu as pltpu
from jax.experimental.pallas import tpu_sc as plsc

N = 10000
E = 320000
D = 128
DE = 16
TT = 3

NC = 2
NS = 16
NW = NC * NS

CHUNK = 128
NSTEPS = E // CHUNK
SPT_MAX = 80

NX = 10112
NF = 30080
RX = NX // NS
RF = NF // NS
NFF = NF * DE // D

BN = 1000
NB = N // BN


def _chunks(total, step):
    out = []
    off = 0
    while off < total:
        out.append((off, min(step, total - off)))
        off += step
    return out


def _tile_steps(wid):
    base = 78 * wid + 2 * jnp.minimum(wid, 2)
    nsteps = jnp.where(wid < 2, 80, 78)
    return base, nsteps


def _node_msg_body(x_ref, m_ref, w0_ref, w1_ref, wx_ref, y_ref):
    xb = x_ref[...]
    a0 = jnp.dot(xb, w0_ref[...], preferred_element_type=jnp.float32)
    a1 = jnp.dot(xb, w1_ref[...], preferred_element_type=jnp.float32)
    x2 = a0 + m_ref[...] * (a1 - a0)
    y_ref[...] = jnp.dot(x2, wx_ref[0], preferred_element_type=jnp.float32)


def _node_msg(x, m, w0t, w1t, wxt):
    return pl.pallas_call(
        _node_msg_body,
        grid=(NB, TT),
        in_specs=[
            pl.BlockSpec((BN, D), lambda i, t: (i, 0)),
            pl.BlockSpec((BN, 1), lambda i, t: (i, 0)),
            pl.BlockSpec((D, D), lambda i, t: (0, 0)),
            pl.BlockSpec((D, D), lambda i, t: (0, 0)),
            pl.BlockSpec((1, D, D), lambda i, t: (t, 0, 0)),
        ],
        out_specs=pl.BlockSpec((BN, D), lambda i, t: (t * NB + i, 0)),
        out_shape=jax.ShapeDtypeStruct((TT * N, D), jnp.float32),
    )(x, m, w0t, w1t, wxt)


_KB = 25


def _ef_pack_body(eft_ref, o_ref):
    a = eft_ref[...]
    o_ref[...] = jnp.swapaxes(a.reshape(DE, _KB, CHUNK), 0, 1)


def _ef_pack(eft):
    return pl.pallas_call(
        _ef_pack_body,
        grid=(NSTEPS // _KB,),
        in_specs=[pl.BlockSpec((DE, _KB * CHUNK), lambda i: (0, i))],
        out_specs=pl.BlockSpec((_KB, DE, CHUNK), lambda i: (i, 0, 0)),
        out_shape=jax.ShapeDtypeStruct((NSTEPS, DE, CHUNK), jnp.float32),
    )(eft)


@functools.partial(
    pl.kernel,
    out_type=jax.ShapeDtypeStruct((NC, NX, D), jnp.float32),
    mesh=plsc.VectorSubcoreMesh(core_axis_name="c", subcore_axis_name="s"),
    compiler_params=pltpu.CompilerParams(use_tc_tiling_on_sc=False, needs_layout_passes=False),
    scratch_types=[
        pltpu.VMEM_SHARED((NX, D), jnp.float32),
        pltpu.VMEM((SPT_MAX, CHUNK), jnp.int32),
        pltpu.VMEM((1, CHUNK), jnp.int32),
        pltpu.VMEM((1, CHUNK), jnp.int32),
        pltpu.VMEM((CHUNK, D), jnp.float32),
        pltpu.VMEM((CHUNK, D), jnp.float32),
        pltpu.SemaphoreType.DMA,
        pltpu.SemaphoreType.DMA,
    ],
)
def _sc_scatter_x(y_hbm, gidx_hbm, dst_hbm, px_hbm,
                  accx, gidx_v, dstA, dstB, rowsA, rowsB, semA, semB):
    cid = lax.axis_index("c")
    sid = lax.axis_index("s")
    wid = cid * NS + sid
    base, nsteps = _tile_steps(wid)

    def _zrow(i, c):
        for j in range(D // 16):
            rowsA[i, pl.ds(j * 16, 16)] = jnp.zeros((16,), jnp.float32)
        return c
    lax.fori_loop(0, CHUNK, _zrow, 0)
    bx = sid * RX
    for off, nr in _chunks(RX, CHUNK):
        pltpu.sync_copy(rowsA.at[pl.ds(0, nr)], accx.at[pl.ds(bx + off, nr)])
    plsc.subcore_barrier()

    pltpu.sync_copy(gidx_hbm.at[pl.ds(base, 78)], gidx_v.at[pl.ds(0, 78)])

    @pl.when(wid < 2)
    def _():
        pltpu.sync_copy(gidx_hbm.at[pl.ds(base + 78, 2)], gidx_v.at[pl.ds(78, 2)])

    def _waitA():
        pltpu.make_async_copy(y_hbm.at[gidx_v.at[0]], rowsA, semA).wait()

    def _waitB():
        pltpu.make_async_copy(y_hbm.at[gidx_v.at[0]], rowsB, semB).wait()

    pltpu.sync_copy(dst_hbm.at[base], dstA.at[0])
    pltpu.async_copy(y_hbm.at[gidx_v.at[0]], rowsA, semA)

    def _pair(jj, c):
        s1 = 2 * jj + 1
        pltpu.sync_copy(dst_hbm.at[base + s1], dstB.at[0])
        pltpu.async_copy(y_hbm.at[gidx_v.at[s1]], rowsB, semB)
        _waitA()
        pltpu.sync_copy(rowsA, accx.at[dstA.at[0]], add=True)
        s2 = 2 * jj + 2
        pltpu.sync_copy(dst_hbm.at[base + s2], dstA.at[0])
        pltpu.async_copy(y_hbm.at[gidx_v.at[s2]], rowsA, semA)
        _waitB()
        pltpu.sync_copy(rowsB, accx.at[dstB.at[0]], add=True)
        return c
    lax.fori_loop(0, nsteps // 2 - 1, _pair, 0, unroll=False)

    pltpu.sync_copy(dst_hbm.at[base + nsteps - 1], dstB.at[0])
    pltpu.async_copy(y_hbm.at[gidx_v.at[nsteps - 1]], rowsB, semB)
    _waitA()
    pltpu.sync_copy(rowsA, accx.at[dstA.at[0]], add=True)
    _waitB()
    pltpu.sync_copy(rowsB, accx.at[dstB.at[0]], add=True)
    plsc.subcore_barrier()

    for off, nr in _chunks(RX, CHUNK):
        pltpu.sync_copy(accx.at[pl.ds(bx + off, nr)], rowsA.at[pl.ds(0, nr)])
        pltpu.sync_copy(rowsA.at[pl.ds(0, nr)], px_hbm.at[cid, pl.ds(bx + off, nr)])


@functools.partial(
    pl.kernel,
    out_type=jax.ShapeDtypeStruct((NC, NFF, D), jnp.float32),
    mesh=plsc.VectorSubcoreMesh(core_axis_name="c", subcore_axis_name="s"),
    compiler_params=pltpu.CompilerParams(use_tc_tiling_on_sc=False, needs_layout_passes=False),
    scratch_types=[
        pltpu.VMEM_SHARED((NF, DE), jnp.float32),
        pltpu.VMEM((SPT_MAX, CHUNK), jnp.int32),
        pltpu.VMEM((DE, CHUNK), jnp.float32),
        pltpu.VMEM((DE, CHUNK), jnp.float32),
        pltpu.VMEM((CHUNK, DE), jnp.float32),
        pltpu.SemaphoreType.DMA,
        pltpu.SemaphoreType.DMA,
    ],
)
def _sc_scatter_f(ef_hbm, sidx_hbm, pf_hbm,
                  accf, sidx_v, efLA, efLB, efS, semA, semB):
    cid = lax.axis_index("c")
    sid = lax.axis_index("s")
    wid = cid * NS + sid
    base, nsteps = _tile_steps(wid)
    iota = lax.iota(jnp.int32, 16)

    def _zrow(i, c):
        efS[i] = jnp.zeros((16,), jnp.float32)
        return c
    lax.fori_loop(0, CHUNK, _zrow, 0)
    bf = sid * RF
    for off, nr in _chunks(RF, CHUNK):
        pltpu.sync_copy(efS.at[pl.ds(0, nr)], accf.at[pl.ds(bf + off, nr)])
    plsc.subcore_barrier()

    pltpu.sync_copy(sidx_hbm.at[pl.ds(base, 78)], sidx_v.at[pl.ds(0, 78)])

    @pl.when(wid < 2)
    def _():
        pltpu.sync_copy(sidx_hbm.at[pl.ds(base + 78, 2)], sidx_v.at[pl.ds(78, 2)])

    def _transpose_into_efS(src_ref):
        def _row(i, c):
            efS[i] = plsc.load_gather(src_ref, [iota, jnp.full((16,), i, jnp.int32)])
            return c
        lax.fori_loop(0, CHUNK, _row, 0)

    def _waitA():
        pltpu.make_async_copy(ef_hbm.at[base], efLA, semA).wait()

    def _waitB():
        pltpu.make_async_copy(ef_hbm.at[base], efLB, semB).wait()

    pltpu.async_copy(ef_hbm.at[base], efLA, semA)

    def _pair(jj, c):
        s1 = 2 * jj + 1
        pltpu.async_copy(ef_hbm.at[base + s1], efLB, semB)
        _waitA()
        _transpose_into_efS(efLA)
        pltpu.sync_copy(efS, accf.at[sidx_v.at[2 * jj]], add=True)
        s2 = 2 * jj + 2
        pltpu.async_copy(ef_hbm.at[base + s2], efLA, semA)
        _waitB()
        _transpose_into_efS(efLB)
        pltpu.sync_copy(efS, accf.at[sidx_v.at[s1]], add=True)
        return c
    lax.fori_loop(0, nsteps // 2 - 1, _pair, 0, unroll=False)

    pltpu.async_copy(ef_hbm.at[base + nsteps - 1], efLB, semB)
    _waitA()
    _transpose_into_efS(efLA)
    pltpu.sync_copy(efS, accf.at[sidx_v.at[nsteps - 2]], add=True)
    _waitB()
    _transpose_into_efS(efLB)
    pltpu.sync_copy(efS, accf.at[sidx_v.at[nsteps - 1]], add=True)
    plsc.subcore_barrier()

    for off, nr in _chunks(RF, CHUNK):
        pltpu.sync_copy(accf.at[pl.ds(bf + off, nr)], efS.at[pl.ds(0, nr)])

        def _fold(j, c):
            for k in range(8):
                efLA[j, pl.ds(16 * k, 16)] = efS[8 * j + k]
            return c
        lax.fori_loop(0, nr // 8, _fold, 0)
        fr = (bf + off) // 8
        pltpu.sync_copy(efLA.at[pl.ds(0, nr // 8)],
                        pf_hbm.at[cid, pl.ds(fr, nr // 8)])


_FT = N * DE // D


def _combine_body(px_ref, pf_ref, wb_ref, o_ref):
    o = px_ref[0, :N] + px_ref[1, :N]
    for t in range(TT):
        s = pf_ref[0, _FT * t:_FT * (t + 1)] + pf_ref[1, _FT * t:_FT * (t + 1)]
        z = jnp.dot(s, wb_ref[t], preferred_element_type=jnp.float32)
        o = o + z.reshape(N, D)
    o_ref[...] = o


def _combine(px, pf, wbig):
    return pl.pallas_call(
        _combine_body,
        out_shape=jax.ShapeDtypeStruct((N, D), jnp.float32),
    )(px, pf, wbig)


def kernel(x, edge_index, edge_feature, node_type, edge_type, W_node, W_msg):
    w0t = W_node[0].T
    w1t = W_node[1].T
    wxt = jnp.transpose(W_msg[:, :, :D], (0, 2, 1))
    wft = jnp.transpose(W_msg[:, :, D:], (0, 2, 1))
    wbig = (jnp.eye(8, dtype=jnp.float32)[None, :, None, :, None]
            * wft[:, None, :, None, :]).reshape(TT, D, 8 * D)
    m = node_type.astype(jnp.float32)[:, None]

    y = _node_msg(x, m, w0t, w1t, wxt)
    efp = _ef_pack(edge_feature.T)

    src = edge_index[0]
    dst = edge_index[1]
    gidx = (edge_type * N + src).reshape(NSTEPS, CHUNK)
    dstp = dst.reshape(NSTEPS, CHUNK)
    sidx = edge_type * N + dst

    px = _sc_scatter_x(y, gidx, dstp)
    sidx_b, px_b = lax.optimization_barrier((sidx.reshape(NSTEPS, CHUNK), px))
    pf = _sc_scatter_f(efp, sidx_b)
    return _combine(px_b, pf, wbig)


# speedup vs baseline: 1.3069x; 1.2998x over previous
import functools

import jax
import jax.numpy as jnp
from jax import lax
from jax.experimental import pallas as pl
from jax.experimental.pallas import tpu as pltpu
from jax.experimental.pallas import tpu_sc as plsc

N = 10000
E = 320000
D = 128
DE = 16
TT = 3

NC = 2
NS = 16
NW = NC * NS

CHUNK = 128
NSTEPS = E // CHUNK
SPT_MAX = 80

NX = 10112
NF = 30080
RX = NX // NS
RF = NF // NS
NFF = NF * DE // D

BN = 1000
NB = N // BN


def _chunks(total, step):
    out = []
    off = 0
    while off < total:
        out.append((off, min(step, total - off)))
        off += step
    return out


def _tile_steps(wid):
    base = 78 * wid + 2 * jnp.minimum(wid, 2)
    nsteps = jnp.where(wid < 2, 80, 78)
    return base, nsteps


def _node_msg_body(x_ref, m_ref, w0_ref, w1_ref, wx_ref, y_ref):
    xb = x_ref[...]
    a0 = jnp.dot(xb, w0_ref[...], preferred_element_type=jnp.float32)
    a1 = jnp.dot(xb, w1_ref[...], preferred_element_type=jnp.float32)
    x2 = a0 + m_ref[...] * (a1 - a0)
    y_ref[...] = jnp.dot(x2, wx_ref[0], preferred_element_type=jnp.float32)


def _node_msg(x, m, w0t, w1t, wxt):
    return pl.pallas_call(
        _node_msg_body,
        grid=(NB, TT),
        in_specs=[
            pl.BlockSpec((BN, D), lambda i, t: (i, 0)),
            pl.BlockSpec((BN, 1), lambda i, t: (i, 0)),
            pl.BlockSpec((D, D), lambda i, t: (0, 0)),
            pl.BlockSpec((D, D), lambda i, t: (0, 0)),
            pl.BlockSpec((1, D, D), lambda i, t: (t, 0, 0)),
        ],
        out_specs=pl.BlockSpec((BN, D), lambda i, t: (t * NB + i, 0)),
        out_shape=jax.ShapeDtypeStruct((TT * N, D), jnp.float32),
    )(x, m, w0t, w1t, wxt)


@functools.partial(
    pl.kernel,
    out_type=jax.ShapeDtypeStruct((NC, NX, D), jnp.float32),
    mesh=plsc.VectorSubcoreMesh(core_axis_name="c", subcore_axis_name="s"),
    compiler_params=pltpu.CompilerParams(use_tc_tiling_on_sc=False),
    scratch_types=[
        pltpu.VMEM_SHARED((NX, D), jnp.float32),
        pltpu.VMEM((SPT_MAX, CHUNK), jnp.int32),
        pltpu.VMEM((1, CHUNK), jnp.int32),
        pltpu.VMEM((1, CHUNK), jnp.int32),
        pltpu.VMEM((CHUNK, D), jnp.float32),
        pltpu.VMEM((CHUNK, D), jnp.float32),
        pltpu.SemaphoreType.DMA,
        pltpu.SemaphoreType.DMA,
    ],
)
def _sc_scatter_x(y_hbm, gidx_hbm, dst_hbm, px_hbm,
                  accx, gidx_v, dstA, dstB, rowsA, rowsB, semA, semB):
    cid = lax.axis_index("c")
    sid = lax.axis_index("s")
    wid = cid * NS + sid
    base, nsteps = _tile_steps(wid)

    def _zrow(i, c):
        for j in range(D // 16):
            rowsA[i, pl.ds(j * 16, 16)] = jnp.zeros((16,), jnp.float32)
        return c
    lax.fori_loop(0, CHUNK, _zrow, 0)
    bx = sid * RX
    for off, nr in _chunks(RX, CHUNK):
        pltpu.sync_copy(rowsA.at[pl.ds(0, nr)], accx.at[pl.ds(bx + off, nr)])
    plsc.subcore_barrier()

    pltpu.sync_copy(gidx_hbm.at[pl.ds(base, 78)], gidx_v.at[pl.ds(0, 78)])

    @pl.when(wid < 2)
    def _():
        pltpu.sync_copy(gidx_hbm.at[pl.ds(base + 78, 2)], gidx_v.at[pl.ds(78, 2)])

    def _waitA():
        pltpu.make_async_copy(y_hbm.at[gidx_v.at[0]], rowsA, semA).wait()

    def _waitB():
        pltpu.make_async_copy(y_hbm.at[gidx_v.at[0]], rowsB, semB).wait()

    pltpu.sync_copy(dst_hbm.at[base], dstA.at[0])
    pltpu.async_copy(y_hbm.at[gidx_v.at[0]], rowsA, semA)

    def _pair(jj, c):
        s1 = 2 * jj + 1
        pltpu.sync_copy(dst_hbm.at[base + s1], dstB.at[0])
        pltpu.async_copy(y_hbm.at[gidx_v.at[s1]], rowsB, semB)
        _waitA()
        pltpu.sync_copy(rowsA, accx.at[dstA.at[0]], add=True)
        s2 = 2 * jj + 2
        pltpu.sync_copy(dst_hbm.at[base + s2], dstA.at[0])
        pltpu.async_copy(y_hbm.at[gidx_v.at[s2]], rowsA, semA)
        _waitB()
        pltpu.sync_copy(rowsB, accx.at[dstB.at[0]], add=True)
        return c
    lax.fori_loop(0, nsteps // 2 - 1, _pair, 0, unroll=False)

    pltpu.sync_copy(dst_hbm.at[base + nsteps - 1], dstB.at[0])
    pltpu.async_copy(y_hbm.at[gidx_v.at[nsteps - 1]], rowsB, semB)
    _waitA()
    pltpu.sync_copy(rowsA, accx.at[dstA.at[0]], add=True)
    _waitB()
    pltpu.sync_copy(rowsB, accx.at[dstB.at[0]], add=True)
    plsc.subcore_barrier()

    for off, nr in _chunks(RX, CHUNK):
        pltpu.sync_copy(accx.at[pl.ds(bx + off, nr)], rowsA.at[pl.ds(0, nr)])
        pltpu.sync_copy(rowsA.at[pl.ds(0, nr)], px_hbm.at[cid, pl.ds(bx + off, nr)])


@functools.partial(
    pl.kernel,
    out_type=jax.ShapeDtypeStruct((NC, NFF, D), jnp.float32),
    mesh=plsc.VectorSubcoreMesh(core_axis_name="c", subcore_axis_name="s"),
    compiler_params=pltpu.CompilerParams(use_tc_tiling_on_sc=False),
    scratch_types=[
        pltpu.VMEM_SHARED((NF, DE), jnp.float32),
        pltpu.VMEM((SPT_MAX, CHUNK), jnp.int32),
        pltpu.VMEM((CHUNK, DE), jnp.float32),
        pltpu.VMEM((CHUNK, DE), jnp.float32),
        pltpu.VMEM((DE, CHUNK), jnp.float32),
        pltpu.SemaphoreType.DMA,
        pltpu.SemaphoreType.DMA,
    ],
)
def _sc_scatter_f(ef_hbm, sidx_hbm, pf_hbm,
                  accf, sidx_v, efA, efB, efF, semA, semB):
    cid = lax.axis_index("c")
    sid = lax.axis_index("s")
    wid = cid * NS + sid
    base, nsteps = _tile_steps(wid)

    def _zrow(i, c):
        efA[i] = jnp.zeros((16,), jnp.float32)
        return c
    lax.fori_loop(0, CHUNK, _zrow, 0)
    bf = sid * RF
    for off, nr in _chunks(RF, CHUNK):
        pltpu.sync_copy(efA.at[pl.ds(0, nr)], accf.at[pl.ds(bf + off, nr)])
    plsc.subcore_barrier()

    pltpu.sync_copy(sidx_hbm.at[pl.ds(base, 78)], sidx_v.at[pl.ds(0, 78)])

    @pl.when(wid < 2)
    def _():
        pltpu.sync_copy(sidx_hbm.at[pl.ds(base + 78, 2)], sidx_v.at[pl.ds(78, 2)])

    def _waitA():
        pltpu.make_async_copy(ef_hbm.at[pl.ds(0, CHUNK)], efA, semA).wait()

    def _waitB():
        pltpu.make_async_copy(ef_hbm.at[pl.ds(0, CHUNK)], efB, semB).wait()

    ebase = base * CHUNK
    pltpu.async_copy(ef_hbm.at[pl.ds(ebase, CHUNK)], efA, semA)

    def _pair(jj, c):
        s1 = 2 * jj + 1
        pltpu.async_copy(ef_hbm.at[pl.ds(ebase + s1 * CHUNK, CHUNK)], efB, semB)
        _waitA()
        pltpu.sync_copy(efA, accf.at[sidx_v.at[2 * jj]], add=True)
        s2 = 2 * jj + 2
        pltpu.async_copy(ef_hbm.at[pl.ds(ebase + s2 * CHUNK, CHUNK)], efA, semA)
        _waitB()
        pltpu.sync_copy(efB, accf.at[sidx_v.at[s1]], add=True)
        return c
    lax.fori_loop(0, nsteps // 2 - 1, _pair, 0, unroll=False)

    pltpu.async_copy(ef_hbm.at[pl.ds(ebase + (nsteps - 1) * CHUNK, CHUNK)], efB, semB)
    _waitA()
    pltpu.sync_copy(efA, accf.at[sidx_v.at[nsteps - 2]], add=True)
    _waitB()
    pltpu.sync_copy(efB, accf.at[sidx_v.at[nsteps - 1]], add=True)
    plsc.subcore_barrier()

    for off, nr in _chunks(RF, CHUNK):
        pltpu.sync_copy(accf.at[pl.ds(bf + off, nr)], efA.at[pl.ds(0, nr)])

        def _fold(j, c):
            for k in range(8):
                efF[j, pl.ds(16 * k, 16)] = efA[8 * j + k]
            return c
        lax.fori_loop(0, nr // 8, _fold, 0)
        fr = (bf + off) // 8
        pltpu.sync_copy(efF.at[pl.ds(0, nr // 8)],
                        pf_hbm.at[cid, pl.ds(fr, nr // 8)])


_FT = N * DE // D


def _combine_body(px_ref, pf_ref, wb_ref, o_ref):
    o = px_ref[0, :N] + px_ref[1, :N]
    for t in range(TT):
        s = pf_ref[0, _FT * t:_FT * (t + 1)] + pf_ref[1, _FT * t:_FT * (t + 1)]
        z = jnp.dot(s, wb_ref[t], preferred_element_type=jnp.float32)
        o = o + z.reshape(N, D)
    o_ref[...] = o


def _combine(px, pf, wbig):
    return pl.pallas_call(
        _combine_body,
        out_shape=jax.ShapeDtypeStruct((N, D), jnp.float32),
    )(px, pf, wbig)


def kernel(x, edge_index, edge_feature, node_type, edge_type, W_node, W_msg):
    w0t = W_node[0].T
    w1t = W_node[1].T
    wxt = jnp.transpose(W_msg[:, :, :D], (0, 2, 1))
    wft = jnp.transpose(W_msg[:, :, D:], (0, 2, 1))
    wbig = (jnp.eye(8, dtype=jnp.float32)[None, :, None, :, None]
            * wft[:, None, :, None, :]).reshape(TT, D, 8 * D)
    m = node_type.astype(jnp.float32)[:, None]

    y = _node_msg(x, m, w0t, w1t, wxt)

    src = edge_index[0]
    dst = edge_index[1]
    gidx = (edge_type * N + src).reshape(NSTEPS, CHUNK)
    dstp = dst.reshape(NSTEPS, CHUNK)
    sidxp = (edge_type * N + dst).reshape(NSTEPS, CHUNK)

    px = _sc_scatter_x(y, gidx, dstp)
    pf = _sc_scatter_f(edge_feature, sidxp)
    return _combine(px, pf, wbig)
